# Initial kernel scaffold; baseline (speedup 1.0000x reference)
import functools
import math

import jax
import jax.numpy as jnp
from jax import lax
from jax.experimental import pallas as pl
from jax.experimental.pallas import tpu as pltpu
from jax.experimental.pallas import tpu_sc as plsc

NC = 2
NS = 16
NW = NC * NS
C = 80

D = 16
F32 = jnp.float32


def _proj_body(h_ref, w_ref, b_ref, q_ref, k_ref, v_ref):
    hd = q_ref.shape[1]
    x = jnp.dot(h_ref[...], w_ref[...], preferred_element_type=F32) + b_ref[...]
    q_ref[...] = x[:, :hd]
    k_ref[...] = x[:, hd:2 * hd]
    v_ref[...] = x[:, 2 * hd:]


def _gather_body(ei, k_h, q_h, v_h, kg, qg, vg, sidx, didx, kb, qb, vb, sem):
    cid = lax.axis_index("c")
    sid = lax.axis_index("s")
    wid = sid * NC + cid
    epw = kg.shape[0] // NW

    def body(t, carry):
        base = wid * epw + t * C
        pltpu.sync_copy(ei.at[0, pl.ds(base, C)], sidx)
        pltpu.sync_copy(ei.at[1, pl.ds(base, C)], didx)
        c1 = pltpu.async_copy(k_h.at[sidx], kb, sem)
        c2 = pltpu.async_copy(q_h.at[didx], qb, sem)
        c3 = pltpu.async_copy(v_h.at[sidx], vb, sem)
        c1.wait()
        c2.wait()
        c3.wait()
        pltpu.sync_copy(kb, kg.at[pl.ds(base, C)])
        pltpu.sync_copy(qb, qg.at[pl.ds(base, C)])
        pltpu.sync_copy(vb, vg.at[pl.ds(base, C)])
        return carry

    lax.fori_loop(0, epw // C, body, 0)


def _edge_body(e_ref, kg_ref, qg_ref, vg_ref, we_ref, be_ref, m8_ref, b16_ref,
               eo_ref, wv_ref, s_ref):
    pe = jnp.dot(e_ref[...], we_ref[...], preferred_element_type=F32) + be_ref[...]
    score = kg_ref[...] * qg_ref[...] * pe * (1.0 / math.sqrt(D))
    eo_ref[...] = score
    t = lax.dot_general(score, m8_ref[...], (((1,), (0,)), ((), ())),
                        precision=lax.Precision.HIGHEST,
                        preferred_element_type=F32)
    s16 = jnp.exp(jnp.clip(t, -5.0, 5.0))
    s_ref[...] = s16
    sb = lax.dot_general(s16, b16_ref[...], (((1,), (0,)), ((), ())),
                         precision=lax.Precision.HIGHEST,
                         preferred_element_type=F32)
    wv_ref[...] = vg_ref[...] * sb


def _scatter_body(ei, wv, s16, zbig, zsmall, acc_out, accz_out,
                  acc, accz, didx, wvb, sb):
    cid = lax.axis_index("c")
    sid = lax.axis_index("s")
    n = acc.shape[0]
    rows = n // NS
    r0 = sid * rows
    pltpu.sync_copy(zbig.at[pl.ds(r0, rows)], acc.at[pl.ds(r0, rows)])
    pltpu.sync_copy(zsmall.at[pl.ds(r0, rows)], accz.at[pl.ds(r0, rows)])
    plsc.subcore_barrier()
    wid = sid * NC + cid
    epw = wv.shape[0] // NW

    def body(t, carry):
        base = wid * epw + t * C
        pltpu.sync_copy(ei.at[1, pl.ds(base, C)], didx)
        pltpu.sync_copy(wv.at[pl.ds(base, C)], wvb)
        pltpu.sync_copy(s16.at[pl.ds(base, C)], sb)
        pltpu.sync_copy(wvb, acc.at[didx], add=True)
        pltpu.sync_copy(sb, accz.at[didx], add=True)
        return carry

    lax.fori_loop(0, epw // C, body, 0)
    plsc.subcore_barrier()
    pltpu.sync_copy(acc.at[pl.ds(r0, rows)], acc_out.at[cid, pl.ds(r0, rows)])
    pltpu.sync_copy(accz.at[pl.ds(r0, rows)], accz_out.at[cid, pl.ds(r0, rows)])


def _final_body(a_ref, az_ref, b16_ref, o_ref):
    wv = a_ref[0] + a_ref[1]
    z16 = az_ref[0] + az_ref[1]
    zb = lax.dot_general(z16, b16_ref[...], (((1,), (0,)), ((), ())),
                         precision=lax.Precision.HIGHEST,
                         preferred_element_type=F32)
    o_ref[...] = wv / (zb + 1e-6)


def kernel(edge_index, h, e, Wq, bq, Wk, bk, Wv, bv, We, be):
    n, in_dim = h.shape
    num_e = e.shape[0]
    hd = Wq.shape[1]
    num_heads = hd // D

    edge_index = edge_index.astype(jnp.int32)
    W = jnp.concatenate([Wq, Wk, Wv], axis=1)
    b = jnp.concatenate([bq, bk, bv])[None, :]

    f = jnp.arange(hd) // D
    m8 = (f[:, None] == jnp.arange(16)[None, :]).astype(F32)
    b16 = (jnp.arange(16)[:, None] == f[None, :]).astype(F32)

    rn = 2000
    Q, K, V = pl.pallas_call(
        _proj_body,
        grid=(n // rn,),
        in_specs=[pl.BlockSpec((rn, in_dim), lambda i: (i, 0)),
                  pl.BlockSpec((in_dim, 3 * hd), lambda i: (0, 0)),
                  pl.BlockSpec((1, 3 * hd), lambda i: (0, 0))],
        out_specs=[pl.BlockSpec((rn, hd), lambda i: (i, 0))] * 3,
        out_shape=[jax.ShapeDtypeStruct((n, hd), F32)] * 3,
    )(h, W, b)

    mesh = plsc.VectorSubcoreMesh(core_axis_name="c", subcore_axis_name="s")
    kg, qg, vg = pl.kernel(
        _gather_body,
        out_type=[jax.ShapeDtypeStruct((num_e, hd), F32)] * 3,
        mesh=mesh,
        scratch_types=[pltpu.VMEM((C,), jnp.int32),
                       pltpu.VMEM((C,), jnp.int32),
                       pltpu.VMEM((C, hd), F32),
                       pltpu.VMEM((C, hd), F32),
                       pltpu.VMEM((C, hd), F32),
                       pltpu.SemaphoreType.DMA],
    )(edge_index, K, Q, V)

    be2 = be[None, :]
    eb = 2000
    eo, wv, s16 = pl.pallas_call(
        _edge_body,
        grid=(num_e // eb,),
        in_specs=[pl.BlockSpec((eb, in_dim), lambda i: (i, 0)),
                  pl.BlockSpec((eb, hd), lambda i: (i, 0)),
                  pl.BlockSpec((eb, hd), lambda i: (i, 0)),
                  pl.BlockSpec((eb, hd), lambda i: (i, 0)),
                  pl.BlockSpec((in_dim, hd), lambda i: (0, 0)),
                  pl.BlockSpec((1, hd), lambda i: (0, 0)),
                  pl.BlockSpec((hd, 16), lambda i: (0, 0)),
                  pl.BlockSpec((16, hd), lambda i: (0, 0))],
        out_specs=[pl.BlockSpec((eb, hd), lambda i: (i, 0)),
                   pl.BlockSpec((eb, hd), lambda i: (i, 0)),
                   pl.BlockSpec((eb, 16), lambda i: (i, 0))],
        out_shape=[jax.ShapeDtypeStruct((num_e, hd), F32),
                   jax.ShapeDtypeStruct((num_e, hd), F32),
                   jax.ShapeDtypeStruct((num_e, 16), F32)],
    )(e, kg, qg, vg, We, be2, m8, b16)

    zbig = jnp.zeros((n, hd), F32)
    zsmall = jnp.zeros((n, 16), F32)
    acc, accz = pl.kernel(
        _scatter_body,
        out_type=[jax.ShapeDtypeStruct((2, n, hd), F32),
                  jax.ShapeDtypeStruct((2, n, 16), F32)],
        mesh=mesh,
        scratch_types=[pltpu.VMEM_SHARED((n, hd), F32),
                       pltpu.VMEM_SHARED((n, 16), F32),
                       pltpu.VMEM((C,), jnp.int32),
                       pltpu.VMEM((C, hd), F32),
                       pltpu.VMEM((C, 16), F32)],
    )(edge_index, wv, s16, zbig, zsmall)

    rb = 2000
    h_out = pl.pallas_call(
        _final_body,
        grid=(n // rb,),
        in_specs=[pl.BlockSpec((2, rb, hd), lambda i: (0, i, 0)),
                  pl.BlockSpec((2, rb, 16), lambda i: (0, i, 0)),
                  pl.BlockSpec((16, hd), lambda i: (0, 0))],
        out_specs=pl.BlockSpec((rb, hd), lambda i: (i, 0)),
        out_shape=jax.ShapeDtypeStruct((n, hd), F32),
    )(acc, accz, b16)

    return h_out.reshape(n, num_heads, D), eo.reshape(num_e, num_heads, D)


# speedup vs baseline: 20.6907x; 20.6907x over previous
import functools
import math

import jax
import jax.numpy as jnp
from jax import lax
from jax.experimental import pallas as pl
from jax.experimental.pallas import tpu as pltpu
from jax.experimental.pallas import tpu_sc as plsc

NC = 2
NS = 16
NW = NC * NS
C = 80
ZL = 128
NH_SPLIT = 5000

D = 16
F32 = jnp.float32


def _proj_body(h_ref, w_ref, b_ref, q_ref, k_ref, v_ref):
    hd = q_ref.shape[1]
    x = jnp.dot(h_ref[...], w_ref[...], preferred_element_type=F32) + b_ref[...]
    q_ref[...] = x[:, :hd]
    k_ref[...] = x[:, hd:2 * hd]
    v_ref[...] = x[:, 2 * hd:]


def _gather_body(src, dst, k_h, q_h, v_h, kg, qg, vg, sidx, didx, kb, qb, vb, sem):
    cid = lax.axis_index("c")
    sid = lax.axis_index("s")
    wid = sid * NC + cid
    epw = kg.shape[0] // NW

    def body(t, carry):
        base = wid * epw + t * C
        pltpu.sync_copy(src.at[pl.ds(base, C)], sidx)
        pltpu.sync_copy(dst.at[pl.ds(base, C)], didx)
        c1 = pltpu.async_copy(k_h.at[sidx], kb, sem)
        c2 = pltpu.async_copy(q_h.at[didx], qb, sem)
        c3 = pltpu.async_copy(v_h.at[sidx], vb, sem)
        c1.wait()
        c2.wait()
        c3.wait()
        pltpu.sync_copy(kb, kg.at[pl.ds(base, C)])
        pltpu.sync_copy(qb, qg.at[pl.ds(base, C)])
        pltpu.sync_copy(vb, vg.at[pl.ds(base, C)])
        return carry

    lax.fori_loop(0, epw // C, body, 0)


def _edge_body(e_ref, kg_ref, qg_ref, vg_ref, we_ref, be_ref, m8_ref, b16_ref,
               eo_ref, wv_ref, s_ref):
    pe = jnp.dot(e_ref[...], we_ref[...], preferred_element_type=F32) + be_ref[...]
    score = kg_ref[...] * qg_ref[...] * pe * (1.0 / math.sqrt(D))
    eo_ref[...] = score
    t = lax.dot_general(score, m8_ref[...], (((1,), (0,)), ((), ())),
                        precision=lax.Precision.HIGHEST,
                        preferred_element_type=F32)
    s16 = jnp.exp(jnp.clip(t, -5.0, 5.0))
    s_ref[...] = s16
    sb = lax.dot_general(s16, b16_ref[...], (((1,), (0,)), ((), ())),
                         precision=lax.Precision.HIGHEST,
                         preferred_element_type=F32)
    wv_ref[...] = vg_ref[...] * sb


def _scatter_body(dst, wv, s16, zbig, zsmall, acc_out, accz_out,
                  acc, accz, didx2, wvb, sb):
    cid = lax.axis_index("c")
    sid = lax.axis_index("s")
    np2 = acc.shape[0]
    rows = np2 // NS
    r0 = sid * rows
    pltpu.sync_copy(zbig, acc.at[pl.ds(r0, rows)])
    pltpu.sync_copy(zsmall, accz.at[pl.ds(r0, rows)])
    plsc.subcore_barrier()

    epw = wv.shape[0] // NS
    EDGES = wv.shape[0]

    def body(t, carry):
        base = sid * epw + t * C
        pltpu.sync_copy(dst.at[pl.ds(cid * EDGES + base, C)], didx2.at[0])
        pltpu.sync_copy(wv.at[pl.ds(base, C)], wvb)
        pltpu.sync_copy(s16.at[pl.ds(base, C)], sb)
        pltpu.sync_copy(wvb, acc.at[didx2.at[0]], add=True)
        pltpu.sync_copy(sb, accz.at[didx2.at[0]], add=True)
        return carry

    lax.fori_loop(0, epw // C, body, 0)
    plsc.subcore_barrier()
    pltpu.sync_copy(acc.at[pl.ds(r0, rows)], acc_out.at[pl.ds(cid * np2 + r0, rows)])
    pltpu.sync_copy(accz.at[pl.ds(r0, rows)], accz_out.at[pl.ds(cid * np2 + r0, rows)])


def _final_body(a_ref, az_ref, b16_ref, o_ref):
    zb = lax.dot_general(az_ref[...], b16_ref[...], (((1,), (0,)), ((), ())),
                         precision=lax.Precision.HIGHEST,
                         preferred_element_type=F32)
    o_ref[...] = a_ref[...] / (zb + 1e-6)


def kernel(edge_index, h, e, Wq, bq, Wk, bk, Wv, bv, We, be):
    n, in_dim = h.shape
    num_e = e.shape[0]
    hd = Wq.shape[1]
    num_heads = hd // D

    edge_index = edge_index.astype(jnp.int32)
    src = edge_index[0]
    dst = edge_index[1]
    W = jnp.concatenate([Wq, Wk, Wv], axis=1)
    b = jnp.concatenate([bq, bk, bv])[None, :]

    f = jnp.arange(hd) // D
    m8 = (f[:, None] == (jnp.arange(ZL) % 16)[None, :]).astype(F32)
    b16 = (jnp.arange(ZL)[:, None] == f[None, :]).astype(F32)

    rn = 2000
    Q, K, V = pl.pallas_call(
        _proj_body,
        grid=(n // rn,),
        in_specs=[pl.BlockSpec((rn, in_dim), lambda i: (i, 0)),
                  pl.BlockSpec((in_dim, 3 * hd), lambda i: (0, 0)),
                  pl.BlockSpec((1, 3 * hd), lambda i: (0, 0))],
        out_specs=[pl.BlockSpec((rn, hd), lambda i: (i, 0))] * 3,
        out_shape=[jax.ShapeDtypeStruct((n, hd), F32)] * 3,
    )(h, W, b)

    mesh = plsc.VectorSubcoreMesh(core_axis_name="c", subcore_axis_name="s")
    kg, qg, vg = pl.kernel(
        _gather_body,
        out_type=[jax.ShapeDtypeStruct((num_e, hd), F32)] * 3,
        mesh=mesh,
        scratch_types=[pltpu.VMEM((C,), jnp.int32),
                       pltpu.VMEM((C,), jnp.int32),
                       pltpu.VMEM((C, hd), F32),
                       pltpu.VMEM((C, hd), F32),
                       pltpu.VMEM((C, hd), F32),
                       pltpu.SemaphoreType.DMA],
    )(src, dst, K, Q, V)

    be2 = be[None, :]
    eb = 2000
    eo, wv, s16 = pl.pallas_call(
        _edge_body,
        grid=(num_e // eb,),
        in_specs=[pl.BlockSpec((eb, in_dim), lambda i: (i, 0)),
                  pl.BlockSpec((eb, hd), lambda i: (i, 0)),
                  pl.BlockSpec((eb, hd), lambda i: (i, 0)),
                  pl.BlockSpec((eb, hd), lambda i: (i, 0)),
                  pl.BlockSpec((in_dim, hd), lambda i: (0, 0)),
                  pl.BlockSpec((1, hd), lambda i: (0, 0)),
                  pl.BlockSpec((hd, ZL), lambda i: (0, 0)),
                  pl.BlockSpec((ZL, hd), lambda i: (0, 0))],
        out_specs=[pl.BlockSpec((eb, hd), lambda i: (i, 0)),
                   pl.BlockSpec((eb, hd), lambda i: (i, 0)),
                   pl.BlockSpec((eb, ZL), lambda i: (i, 0))],
        out_shape=[jax.ShapeDtypeStruct((num_e, hd), F32),
                   jax.ShapeDtypeStruct((num_e, hd), F32),
                   jax.ShapeDtypeStruct((num_e, ZL), F32)],
    )(e, kg, qg, vg, We, be2, m8, b16)

    global NH_SPLIT
    nh = -(-n // NC)
    NH_SPLIT = nh
    np2 = -(-nh // (NS * 64)) * (NS * 64)
    if np2 == nh:
        np2 += NS * 64
    rows = np2 // NS
    core = jnp.arange(NC)[:, None]
    dl = dst[None, :] - core * nh
    dstf = jnp.where((dl >= 0) & (dl < nh), dl, np2 - 1).astype(jnp.int32).reshape(-1)
    zbig = jnp.zeros((rows, hd), F32)
    zsmall = jnp.zeros((rows, ZL), F32)
    acc, accz = pl.kernel(
        _scatter_body,
        out_type=[jax.ShapeDtypeStruct((NC * np2, hd), F32),
                  jax.ShapeDtypeStruct((NC * np2, ZL), F32)],
        mesh=mesh,
        scratch_types=[pltpu.VMEM_SHARED((np2, hd), F32),
                       pltpu.VMEM_SHARED((np2, ZL), F32),
                       pltpu.VMEM((8, C), jnp.int32),
                       pltpu.VMEM((C, hd), F32),
                       pltpu.VMEM((C, ZL), F32)],
    )(dstf, wv, s16, zbig, zsmall)

    rb = 1024
    h_out = pl.pallas_call(
        _final_body,
        grid=(NC * np2 // rb,),
        in_specs=[pl.BlockSpec((rb, hd), lambda i: (i, 0)),
                  pl.BlockSpec((rb, ZL), lambda i: (i, 0)),
                  pl.BlockSpec((ZL, hd), lambda i: (0, 0))],
        out_specs=pl.BlockSpec((rb, hd), lambda i: (i, 0)),
        out_shape=jax.ShapeDtypeStruct((NC * np2, hd), F32),
    )(acc, accz, b16)

    h_out = jnp.concatenate(
        [lax.dynamic_slice_in_dim(h_out, c * np2, nh)[:min(nh, n - c * nh)]
         for c in range(NC)], axis=0)
    return h_out.reshape(n, num_heads, D), eo.reshape(num_e, num_heads, D)
